# baseline (speedup 1.0000x reference)
import jax
import jax.numpy as jnp
from jax.experimental import pallas as pl
from jax.experimental.pallas import tpu as pltpu

B = 4096
I = 4096
O = 4096
NCORES = 2
BR = 512
KW = (O // NCORES) // BR
KX = (B // NCORES) // BR


def _wsum_body(w_ref, o_ref):
    k = pl.program_id(1)
    part = jnp.sum(w_ref[...], axis=0, keepdims=True)

    @pl.when(k == 0)
    def _init():
        o_ref[...] = part[None]

    @pl.when(k > 0)
    def _acc():
        o_ref[...] += part[None]


def _out_body(x_ref, ws_ref, b_ref, o_ref):
    wsum = jnp.sum(ws_ref[...], axis=0)
    part = jnp.sum(x_ref[...] * wsum, axis=1, keepdims=True)
    o_ref[...] = part + jnp.sum(b_ref[...])


def kernel(x, W, b):
    wpart = pl.pallas_call(
        _wsum_body,
        grid=(NCORES, KW),
        in_specs=[pl.BlockSpec((BR, I), lambda c, k: (c * KW + k, 0))],
        out_specs=pl.BlockSpec((1, 1, I), lambda c, k: (c, 0, 0)),
        out_shape=jax.ShapeDtypeStruct((NCORES, 1, I), jnp.float32),
        compiler_params=pltpu.CompilerParams(
            dimension_semantics=("parallel", "arbitrary"),
        ),
    )(W)

    out = pl.pallas_call(
        _out_body,
        grid=(NCORES, KX),
        in_specs=[
            pl.BlockSpec((BR, I), lambda c, k: (c * KX + k, 0)),
            pl.BlockSpec((NCORES, 1, I), lambda c, k: (0, 0, 0)),
            pl.BlockSpec((1, I), lambda c, k: (0, 0)),
        ],
        out_specs=pl.BlockSpec((BR, 1), lambda c, k: (c * KX + k, 0)),
        out_shape=jax.ShapeDtypeStruct((B, 1), jnp.float32),
        compiler_params=pltpu.CompilerParams(
            dimension_semantics=("parallel", "arbitrary"),
        ),
    )(x, wpart, b.reshape(1, I))
    return out
